# baseline (speedup 1.0000x reference)
import jax
import jax.numpy as jnp
from jax.experimental import pallas as pl
from jax.experimental.pallas import tpu as pltpu

N = 4096
D_IN = 512
D_HID = 256
N_CLS = 32
P_LEN = N * (N - 1) // 2
TI = 256
NT = N // TI
WW = N + 128
VEC_PAD_LEN = P_LEN + 1 + WW + 128
SD = 4


def _build_body(vec_ref, sub_ref, x_ref, w1_ref,
                p_ref, a_ref, deg_ref, z_ref,
                lraw, s_buf, subt, stp, stpt, sta, stat, stpd, stad,
                acc_tile, colacc,
                load_sems, sub_sems, st_sems, diag_sems):
    I = pl.program_id(0)
    i0 = I * TI

    @pl.when(I == 0)
    def _():
        colacc[...] = jnp.zeros_like(colacc)

    def _issue_loads(Iw, buf):
        iw0 = Iw * TI

        def body(r, _):
            i = iw0 + r
            off_w = i * (N - 1) - (i * (i - 1)) // 2 - i + iw0
            q = off_w // 128
            pltpu.make_async_copy(vec_ref.at[pl.ds(q * 128, WW)],
                                  lraw.at[buf, r], load_sems.at[buf]).start()
            return 0

        jax.lax.fori_loop(0, TI, body, 0, unroll=16)

    def _issue_sub(J, s):
        pltpu.make_async_copy(
            sub_ref.at[pl.ds(i0, TI), pl.ds(J * TI, TI)],
            subt.at[s], sub_sems.at[s]).start()

    @pl.when(I == 0)
    def _():
        _issue_loads(0, 0)

    @pl.when(I + 1 < NT)
    def _():
        _issue_loads(I + 1, (I + 1) & 1)

    for dj in range(3):
        @pl.when(I + dj < NT)
        def _():
            _issue_sub(I + dj, (I + dj) & (SD - 1))

    z_ref[...] = jnp.dot(x_ref[...].astype(jnp.bfloat16), w1_ref[...],
                         preferred_element_type=jnp.float32
                         ).astype(jnp.bfloat16)

    def _wait_load(r, _):
        pltpu.make_async_copy(vec_ref.at[pl.ds(0, WW)], lraw.at[I & 1, 0],
                              load_sems.at[I & 1]).wait()
        return 0

    jax.lax.fori_loop(0, TI, _wait_load, 0, unroll=16)

    r2 = jax.lax.broadcasted_iota(jnp.int32, (TI, 1), 0)
    i2 = i0 + r2
    off_w2 = i2 * (N - 1) - (i2 * (i2 - 1)) // 2 - i2 + i0
    sh = off_w2 & 127
    cur = lraw[I & 1]
    for b in (64, 32, 16, 8, 4, 2, 1):
        rolled = pltpu.roll(cur, WW - b, axis=1)
        cur = jnp.where((sh & b) != 0, rolled, cur)
    s_buf[...] = cur

    rl = jax.lax.broadcasted_iota(jnp.int32, (TI, TI), 0)
    cl = jax.lax.broadcasted_iota(jnp.int32, (TI, TI), 1)
    eye = jnp.where(rl == cl, 1.0, 0.0)
    ones8 = jnp.ones((8, TI), jnp.float32)

    pltpu.make_async_copy(
        sub_ref.at[pl.ds(i0, TI), pl.ds(0, TI)],
        subt.at[I & (SD - 1)], sub_sems.at[I & (SD - 1)]).wait()
    td = s_buf[:, pl.ds(0, TI)]
    md = jnp.where(cl > rl, td, 0.0)
    pud = jax.nn.sigmoid(md + md.T)
    affd = pud * subt[I & (SD - 1)] + eye

    @pl.when(I > 0)
    def _():
        pltpu.make_async_copy(stpd, p_ref.at[pl.ds(0, TI), pl.ds(0, TI)], diag_sems.at[0]).wait()
        pltpu.make_async_copy(stad, a_ref.at[pl.ds(0, TI), pl.ds(0, TI)], diag_sems.at[1]).wait()

    stpd[...] = pud
    stad[...] = affd.astype(jnp.bfloat16)
    pltpu.make_async_copy(stpd, p_ref.at[pl.ds(i0, TI), pl.ds(i0, TI)],
                          diag_sems.at[0]).start()
    pltpu.make_async_copy(stad, a_ref.at[pl.ds(i0, TI), pl.ds(i0, TI)],
                          diag_sems.at[1]).start()
    acc_tile[...] = affd

    def _tile(J, carry):
        s = J & (SD - 1)
        j0 = J * TI

        @pl.when(J + 2 < NT)
        def _():
            _issue_sub(J + 2, (J + 2) & (SD - 1))

        pltpu.make_async_copy(
            sub_ref.at[pl.ds(i0, TI), pl.ds(0, TI)],
            subt.at[s], sub_sems.at[s]).wait()

        t = s_buf[:, pl.ds((J - I) * TI, TI)]
        pu = jax.nn.sigmoid(t)
        put = pu.T
        aff = pu * subt[s]
        af = aff.astype(jnp.bfloat16)
        aft = aff.T.astype(jnp.bfloat16)

        @pl.when(J >= I + 1 + SD)
        def _():
            pltpu.make_async_copy(stp.at[s], p_ref.at[pl.ds(0, TI), pl.ds(0, TI)], st_sems.at[0, s]).wait()
            pltpu.make_async_copy(stpt.at[s], p_ref.at[pl.ds(0, TI), pl.ds(0, TI)], st_sems.at[1, s]).wait()
            pltpu.make_async_copy(sta.at[s], a_ref.at[pl.ds(0, TI), pl.ds(0, TI)], st_sems.at[2, s]).wait()
            pltpu.make_async_copy(stat.at[s], a_ref.at[pl.ds(0, TI), pl.ds(0, TI)], st_sems.at[3, s]).wait()

        stp[s] = pu
        stpt[s] = put
        sta[s] = af
        stat[s] = aft
        pltpu.make_async_copy(stp.at[s], p_ref.at[pl.ds(i0, TI), pl.ds(j0, TI)],
                              st_sems.at[0, s]).start()
        pltpu.make_async_copy(stpt.at[s], p_ref.at[pl.ds(j0, TI), pl.ds(i0, TI)],
                              st_sems.at[1, s]).start()
        pltpu.make_async_copy(sta.at[s], a_ref.at[pl.ds(i0, TI), pl.ds(j0, TI)],
                              st_sems.at[2, s]).start()
        pltpu.make_async_copy(stat.at[s], a_ref.at[pl.ds(j0, TI), pl.ds(i0, TI)],
                              st_sems.at[3, s]).start()

        acc_tile[...] += aff
        cs = jnp.dot(ones8, aff, preferred_element_type=jnp.float32)
        cprev = colacc[pl.ds(J, 1)]
        colacc[pl.ds(J, 1)] = cprev + cs[0:1, :].reshape(1, 1, TI)
        return carry

    jax.lax.fori_loop(I + 1, NT, _tile, 0)

    for jd in range(NT - SD, NT):
        @pl.when(jd >= I + 1)
        def _():
            s = jd & (SD - 1)
            pltpu.make_async_copy(stp.at[s], p_ref.at[pl.ds(0, TI), pl.ds(0, TI)], st_sems.at[0, s]).wait()
            pltpu.make_async_copy(stpt.at[s], p_ref.at[pl.ds(0, TI), pl.ds(0, TI)], st_sems.at[1, s]).wait()
            pltpu.make_async_copy(sta.at[s], a_ref.at[pl.ds(0, TI), pl.ds(0, TI)], st_sems.at[2, s]).wait()
            pltpu.make_async_copy(stat.at[s], a_ref.at[pl.ds(0, TI), pl.ds(0, TI)], st_sems.at[3, s]).wait()

    @pl.when(I == NT - 1)
    def _():
        pltpu.make_async_copy(stpd, p_ref.at[pl.ds(0, TI), pl.ds(0, TI)], diag_sems.at[0]).wait()
        pltpu.make_async_copy(stad, a_ref.at[pl.ds(0, TI), pl.ds(0, TI)], diag_sems.at[1]).wait()

    rs = jnp.dot(acc_tile[...], ones8.T,
                 preferred_element_type=jnp.float32)[:, 0:1]
    deg_ref[...] = (rs.reshape(1, TI) + colacc[pl.ds(I, 1)].reshape(1, TI)
                    ).reshape(1, 1, TI)


def _build_p_and_a(P_vec, sub_adj, x, W1):
    vecpad = jnp.zeros((VEC_PAD_LEN,), jnp.float32).at[1:P_LEN + 1].set(P_vec)
    return pl.pallas_call(
        _build_body,
        grid=(NT,),
        in_specs=[pl.BlockSpec(memory_space=pltpu.MemorySpace.HBM),
                  pl.BlockSpec(memory_space=pltpu.MemorySpace.HBM),
                  pl.BlockSpec((TI, D_IN), lambda i: (i, 0)),
                  pl.BlockSpec((D_IN, D_HID), lambda i: (0, 0))],
        out_specs=[pl.BlockSpec(memory_space=pltpu.MemorySpace.HBM),
                   pl.BlockSpec(memory_space=pltpu.MemorySpace.HBM),
                   pl.BlockSpec((1, 1, TI), lambda i: (i, 0, 0)),
                   pl.BlockSpec((TI, D_HID), lambda i: (i, 0))],
        out_shape=[jax.ShapeDtypeStruct((N, N), jnp.float32),
                   jax.ShapeDtypeStruct((N, N), jnp.bfloat16),
                   jax.ShapeDtypeStruct((NT, 1, TI), jnp.float32),
                   jax.ShapeDtypeStruct((N, D_HID), jnp.bfloat16)],
        scratch_shapes=[
            pltpu.VMEM((2, TI, WW), jnp.float32),
            pltpu.VMEM((TI, WW), jnp.float32),
            pltpu.VMEM((SD, TI, TI), jnp.float32),
            pltpu.VMEM((SD, TI, TI), jnp.float32),
            pltpu.VMEM((SD, TI, TI), jnp.float32),
            pltpu.VMEM((SD, TI, TI), jnp.bfloat16),
            pltpu.VMEM((SD, TI, TI), jnp.bfloat16),
            pltpu.VMEM((TI, TI), jnp.float32),
            pltpu.VMEM((TI, TI), jnp.bfloat16),
            pltpu.VMEM((TI, TI), jnp.float32),
            pltpu.VMEM((NT, 1, TI), jnp.float32),
            pltpu.SemaphoreType.DMA((2,)),
            pltpu.SemaphoreType.DMA((SD,)),
            pltpu.SemaphoreType.DMA((4, SD)),
            pltpu.SemaphoreType.DMA((2,)),
        ],
    )(vecpad, sub_adj, x, W1)


def _layer1_body(a_ref, z_ref, dcol_ref, b1_ref, w2_ref, dinv_ref, g_ref,
                 zs_ref):
    @pl.when(pl.program_id(0) == 0)
    def _():
        zs_ref[...] = (z_ref[...].astype(jnp.float32)
                       * dcol_ref[...]).astype(jnp.bfloat16)

    y = jnp.dot(a_ref[...], zs_ref[...], preferred_element_type=jnp.float32)
    d = dinv_ref[...].reshape(TI, 1)
    h = jnp.maximum(y * d + b1_ref[...], 0.0)
    g = jnp.dot(h, w2_ref[...], preferred_element_type=jnp.float32)
    g_ref[...] = (g * d).astype(jnp.bfloat16)


def _layer2_body(a_ref, g_ref, b2_ref, dinv_ref, out_ref):
    y = jnp.dot(a_ref[...], g_ref[...], preferred_element_type=jnp.float32)
    out_ref[...] = y * dinv_ref[...].reshape(TI, 1) + b2_ref[...]


def kernel(x, P_vec, sub_adj, W1, b1, W2, b2):
    P_used, A, deg_parts, zp = _build_p_and_a(P_vec, sub_adj, x,
                                              W1.astype(jnp.bfloat16))
    deg = deg_parts.reshape(N)
    dinv = jax.lax.rsqrt(deg)
    dcol = dinv.reshape(N, 1)

    g = pl.pallas_call(
        _layer1_body,
        grid=(NT,),
        in_specs=[
            pl.BlockSpec((TI, N), lambda i: (i, 0)),
            pl.BlockSpec((N, D_HID), lambda i: (0, 0)),
            pl.BlockSpec((N, 1), lambda i: (0, 0)),
            pl.BlockSpec((1, D_HID), lambda i: (0, 0)),
            pl.BlockSpec((D_HID, N_CLS), lambda i: (0, 0)),
            pl.BlockSpec((TI,), lambda i: (i,)),
        ],
        out_specs=pl.BlockSpec((TI, N_CLS), lambda i: (i, 0)),
        out_shape=jax.ShapeDtypeStruct((N, N_CLS), jnp.bfloat16),
        scratch_shapes=[pltpu.VMEM((N, D_HID), jnp.bfloat16)],
    )(A, zp, dcol, b1.reshape(1, D_HID), W2, dinv)

    out = pl.pallas_call(
        _layer2_body,
        grid=(NT,),
        in_specs=[
            pl.BlockSpec((TI, N), lambda i: (i, 0)),
            pl.BlockSpec((N, N_CLS), lambda i: (0, 0)),
            pl.BlockSpec((1, N_CLS), lambda i: (0, 0)),
            pl.BlockSpec((TI,), lambda i: (i,)),
        ],
        out_specs=pl.BlockSpec((TI, N_CLS), lambda i: (i, 0)),
        out_shape=jax.ShapeDtypeStruct((N, N_CLS), jnp.float32),
    )(A, g, b2.reshape(1, N_CLS), dinv)

    return (out, P_used)


# speedup vs baseline: 1.2096x; 1.2096x over previous
import jax
import jax.numpy as jnp
from jax.experimental import pallas as pl
from jax.experimental.pallas import tpu as pltpu

N = 4096
D_IN = 512
D_HID = 256
N_CLS = 32
P_LEN = N * (N - 1) // 2
TI = 256
NT = N // TI
WW = N + 128
VEC_PAD_LEN = P_LEN + 1 + WW + 128
SD = 4


def _build_body(vec_ref, sub_ref, x_ref, w1_ref,
                p_ref, a_ref, deg_ref, z_ref,
                lraw, s_buf, subt, stp, stpt, sta, stat, stpd, stad,
                acc_tile, colacc,
                load_sems, sub_sems, st_sems, diag_sems):
    I = pl.program_id(0)
    i0 = I * TI

    @pl.when(I == 0)
    def _():
        colacc[...] = jnp.zeros_like(colacc)

    def _issue_loads(Iw, buf):
        iw0 = Iw * TI

        def body(r, _):
            i = iw0 + r
            off_w = i * (N - 1) - (i * (i - 1)) // 2 - i + iw0
            q = off_w // 128
            pltpu.make_async_copy(vec_ref.at[pl.ds(q * 128, WW)],
                                  lraw.at[buf, r], load_sems.at[buf]).start()
            return 0

        jax.lax.fori_loop(0, TI, body, 0, unroll=16)

    def _issue_sub(J, s):
        pltpu.make_async_copy(
            sub_ref.at[pl.ds(i0, TI), pl.ds(J * TI, TI)],
            subt.at[s], sub_sems.at[s]).start()

    @pl.when(I == 0)
    def _():
        _issue_loads(0, 0)

    @pl.when(I + 1 < NT)
    def _():
        _issue_loads(I + 1, (I + 1) & 1)

    for dj in range(3):
        @pl.when(I + dj < NT)
        def _():
            _issue_sub(I + dj, (I + dj) & (SD - 1))

    z_ref[...] = jnp.dot(x_ref[...].astype(jnp.bfloat16), w1_ref[...],
                         preferred_element_type=jnp.float32
                         ).astype(jnp.bfloat16)

    def _wait_load(r, _):
        pltpu.make_async_copy(vec_ref.at[pl.ds(0, WW)], lraw.at[I & 1, 0],
                              load_sems.at[I & 1]).wait()
        return 0

    jax.lax.fori_loop(0, TI, _wait_load, 0, unroll=16)

    r2 = jax.lax.broadcasted_iota(jnp.int32, (TI, 1), 0)
    i2 = i0 + r2
    off_w2 = i2 * (N - 1) - (i2 * (i2 - 1)) // 2 - i2 + i0
    sh = off_w2 & 127
    cur = lraw[I & 1].astype(jnp.bfloat16)
    for b in (64, 32, 16, 8, 4, 2, 1):
        rolled = pltpu.roll(cur, WW - b, axis=1)
        cur = jnp.where((sh & b) != 0, rolled, cur)
    s_buf[...] = cur

    rl = jax.lax.broadcasted_iota(jnp.int32, (TI, TI), 0)
    cl = jax.lax.broadcasted_iota(jnp.int32, (TI, TI), 1)
    eye = jnp.where(rl == cl, 1.0, 0.0)
    ones8 = jnp.ones((8, TI), jnp.float32)

    pltpu.make_async_copy(
        sub_ref.at[pl.ds(i0, TI), pl.ds(0, TI)],
        subt.at[I & (SD - 1)], sub_sems.at[I & (SD - 1)]).wait()
    td = s_buf[:, pl.ds(0, TI)].astype(jnp.float32)
    md = jnp.where(cl > rl, td, 0.0)
    pud = jax.nn.sigmoid(md + md.T)
    affd = pud * subt[I & (SD - 1)] + eye

    @pl.when(I > 0)
    def _():
        pltpu.make_async_copy(stpd, p_ref.at[pl.ds(0, TI), pl.ds(0, TI)], diag_sems.at[0]).wait()
        pltpu.make_async_copy(stad, a_ref.at[pl.ds(0, TI), pl.ds(0, TI)], diag_sems.at[1]).wait()

    stpd[...] = pud
    stad[...] = affd.astype(jnp.bfloat16)
    pltpu.make_async_copy(stpd, p_ref.at[pl.ds(i0, TI), pl.ds(i0, TI)],
                          diag_sems.at[0]).start()
    pltpu.make_async_copy(stad, a_ref.at[pl.ds(i0, TI), pl.ds(i0, TI)],
                          diag_sems.at[1]).start()
    acc_tile[...] = affd

    def _tile(J, carry):
        s = J & (SD - 1)
        j0 = J * TI

        @pl.when(J + 2 < NT)
        def _():
            _issue_sub(J + 2, (J + 2) & (SD - 1))

        pltpu.make_async_copy(
            sub_ref.at[pl.ds(i0, TI), pl.ds(0, TI)],
            subt.at[s], sub_sems.at[s]).wait()

        t = s_buf[:, pl.ds((J - I) * TI, TI)].astype(jnp.float32)
        pu = jax.nn.sigmoid(t)
        put = pu.T
        aff = pu * subt[s]
        af = aff.astype(jnp.bfloat16)
        aft = aff.T.astype(jnp.bfloat16)

        @pl.when(J >= I + 1 + SD)
        def _():
            pltpu.make_async_copy(stp.at[s], p_ref.at[pl.ds(0, TI), pl.ds(0, TI)], st_sems.at[0, s]).wait()
            pltpu.make_async_copy(stpt.at[s], p_ref.at[pl.ds(0, TI), pl.ds(0, TI)], st_sems.at[1, s]).wait()
            pltpu.make_async_copy(sta.at[s], a_ref.at[pl.ds(0, TI), pl.ds(0, TI)], st_sems.at[2, s]).wait()
            pltpu.make_async_copy(stat.at[s], a_ref.at[pl.ds(0, TI), pl.ds(0, TI)], st_sems.at[3, s]).wait()

        stp[s] = pu
        stpt[s] = put
        sta[s] = af
        stat[s] = aft
        pltpu.make_async_copy(stp.at[s], p_ref.at[pl.ds(i0, TI), pl.ds(j0, TI)],
                              st_sems.at[0, s]).start()
        pltpu.make_async_copy(stpt.at[s], p_ref.at[pl.ds(j0, TI), pl.ds(i0, TI)],
                              st_sems.at[1, s]).start()
        pltpu.make_async_copy(sta.at[s], a_ref.at[pl.ds(i0, TI), pl.ds(j0, TI)],
                              st_sems.at[2, s]).start()
        pltpu.make_async_copy(stat.at[s], a_ref.at[pl.ds(j0, TI), pl.ds(i0, TI)],
                              st_sems.at[3, s]).start()

        acc_tile[...] += aff
        cs = jnp.dot(ones8, aff, preferred_element_type=jnp.float32)
        cprev = colacc[pl.ds(J, 1)]
        colacc[pl.ds(J, 1)] = cprev + cs[0:1, :].reshape(1, 1, TI)
        return carry

    jax.lax.fori_loop(I + 1, NT, _tile, 0)

    for jd in range(NT - SD, NT):
        @pl.when(jd >= I + 1)
        def _():
            s = jd & (SD - 1)
            pltpu.make_async_copy(stp.at[s], p_ref.at[pl.ds(0, TI), pl.ds(0, TI)], st_sems.at[0, s]).wait()
            pltpu.make_async_copy(stpt.at[s], p_ref.at[pl.ds(0, TI), pl.ds(0, TI)], st_sems.at[1, s]).wait()
            pltpu.make_async_copy(sta.at[s], a_ref.at[pl.ds(0, TI), pl.ds(0, TI)], st_sems.at[2, s]).wait()
            pltpu.make_async_copy(stat.at[s], a_ref.at[pl.ds(0, TI), pl.ds(0, TI)], st_sems.at[3, s]).wait()

    @pl.when(I == NT - 1)
    def _():
        pltpu.make_async_copy(stpd, p_ref.at[pl.ds(0, TI), pl.ds(0, TI)], diag_sems.at[0]).wait()
        pltpu.make_async_copy(stad, a_ref.at[pl.ds(0, TI), pl.ds(0, TI)], diag_sems.at[1]).wait()

    rs = jnp.dot(acc_tile[...], ones8.T,
                 preferred_element_type=jnp.float32)[:, 0:1]
    deg_ref[...] = (rs.reshape(1, TI) + colacc[pl.ds(I, 1)].reshape(1, TI)
                    ).reshape(1, 1, TI)


def _build_p_and_a(P_vec, sub_adj, x, W1):
    vecpad = jnp.zeros((VEC_PAD_LEN,), jnp.float32).at[1:P_LEN + 1].set(P_vec)
    return pl.pallas_call(
        _build_body,
        grid=(NT,),
        in_specs=[pl.BlockSpec(memory_space=pltpu.MemorySpace.HBM),
                  pl.BlockSpec(memory_space=pltpu.MemorySpace.HBM),
                  pl.BlockSpec((TI, D_IN), lambda i: (i, 0)),
                  pl.BlockSpec((D_IN, D_HID), lambda i: (0, 0))],
        out_specs=[pl.BlockSpec(memory_space=pltpu.MemorySpace.HBM),
                   pl.BlockSpec(memory_space=pltpu.MemorySpace.HBM),
                   pl.BlockSpec((1, 1, TI), lambda i: (i, 0, 0)),
                   pl.BlockSpec((TI, D_HID), lambda i: (i, 0))],
        out_shape=[jax.ShapeDtypeStruct((N, N), jnp.float32),
                   jax.ShapeDtypeStruct((N, N), jnp.bfloat16),
                   jax.ShapeDtypeStruct((NT, 1, TI), jnp.float32),
                   jax.ShapeDtypeStruct((N, D_HID), jnp.bfloat16)],
        scratch_shapes=[
            pltpu.VMEM((2, TI, WW), jnp.float32),
            pltpu.VMEM((TI, WW), jnp.bfloat16),
            pltpu.VMEM((SD, TI, TI), jnp.float32),
            pltpu.VMEM((SD, TI, TI), jnp.float32),
            pltpu.VMEM((SD, TI, TI), jnp.float32),
            pltpu.VMEM((SD, TI, TI), jnp.bfloat16),
            pltpu.VMEM((SD, TI, TI), jnp.bfloat16),
            pltpu.VMEM((TI, TI), jnp.float32),
            pltpu.VMEM((TI, TI), jnp.bfloat16),
            pltpu.VMEM((TI, TI), jnp.float32),
            pltpu.VMEM((NT, 1, TI), jnp.float32),
            pltpu.SemaphoreType.DMA((2,)),
            pltpu.SemaphoreType.DMA((SD,)),
            pltpu.SemaphoreType.DMA((4, SD)),
            pltpu.SemaphoreType.DMA((2,)),
        ],
    )(vecpad, sub_adj, x, W1)


def _layer1_body(a_ref, z_ref, dcol_ref, b1_ref, w2_ref, dinv_ref, g_ref,
                 zs_ref):
    @pl.when(pl.program_id(0) == 0)
    def _():
        zs_ref[...] = (z_ref[...].astype(jnp.float32)
                       * dcol_ref[...]).astype(jnp.bfloat16)

    y = jnp.dot(a_ref[...], zs_ref[...], preferred_element_type=jnp.float32)
    d = dinv_ref[...].reshape(TI, 1)
    h = jnp.maximum(y * d + b1_ref[...], 0.0)
    g = jnp.dot(h, w2_ref[...], preferred_element_type=jnp.float32)
    g_ref[...] = (g * d).astype(jnp.bfloat16)


def _layer2_body(a_ref, g_ref, b2_ref, dinv_ref, out_ref):
    y = jnp.dot(a_ref[...], g_ref[...], preferred_element_type=jnp.float32)
    out_ref[...] = y * dinv_ref[...].reshape(TI, 1) + b2_ref[...]


def kernel(x, P_vec, sub_adj, W1, b1, W2, b2):
    P_used, A, deg_parts, zp = _build_p_and_a(P_vec, sub_adj, x,
                                              W1.astype(jnp.bfloat16))
    deg = deg_parts.reshape(N)
    dinv = jax.lax.rsqrt(deg)
    dcol = dinv.reshape(N, 1)

    g = pl.pallas_call(
        _layer1_body,
        grid=(NT,),
        in_specs=[
            pl.BlockSpec((TI, N), lambda i: (i, 0)),
            pl.BlockSpec((N, D_HID), lambda i: (0, 0)),
            pl.BlockSpec((N, 1), lambda i: (0, 0)),
            pl.BlockSpec((1, D_HID), lambda i: (0, 0)),
            pl.BlockSpec((D_HID, N_CLS), lambda i: (0, 0)),
            pl.BlockSpec((TI,), lambda i: (i,)),
        ],
        out_specs=pl.BlockSpec((TI, N_CLS), lambda i: (i, 0)),
        out_shape=jax.ShapeDtypeStruct((N, N_CLS), jnp.bfloat16),
        scratch_shapes=[pltpu.VMEM((N, D_HID), jnp.bfloat16)],
    )(A, zp, dcol, b1.reshape(1, D_HID), W2, dinv)

    out = pl.pallas_call(
        _layer2_body,
        grid=(NT,),
        in_specs=[
            pl.BlockSpec((TI, N), lambda i: (i, 0)),
            pl.BlockSpec((N, N_CLS), lambda i: (0, 0)),
            pl.BlockSpec((1, N_CLS), lambda i: (0, 0)),
            pl.BlockSpec((TI,), lambda i: (i,)),
        ],
        out_specs=pl.BlockSpec((TI, N_CLS), lambda i: (i, 0)),
        out_shape=jax.ShapeDtypeStruct((N, N_CLS), jnp.float32),
    )(A, g, b2.reshape(1, N_CLS), dinv)

    return (out, P_used)


# speedup vs baseline: 1.2115x; 1.0015x over previous
import jax
import jax.numpy as jnp
from jax.experimental import pallas as pl
from jax.experimental.pallas import tpu as pltpu

N = 4096
D_IN = 512
D_HID = 256
N_CLS = 32
P_LEN = N * (N - 1) // 2
TI = 256
NT = N // TI
WW = N + 128
VEC_PAD_LEN = P_LEN + 1 + WW + 128
SD = 4


def _build_body(vec_ref, sub_ref, x_ref, w1_ref,
                p_ref, a_ref, deg_ref, z_ref,
                lraw, s_buf, subt, stp, stpt, sta, stat, stpd, stad,
                acc_tile, colacc,
                load_sems, sub_sems, st_sems, diag_sems):
    I = pl.program_id(0)
    i0 = I * TI

    @pl.when(I == 0)
    def _():
        colacc[...] = jnp.zeros_like(colacc)

    def _issue_loads(Iw, buf):
        iw0 = Iw * TI

        def body(r, _):
            i = iw0 + r
            off_w = i * (N - 1) - (i * (i - 1)) // 2 - i + iw0
            q = off_w // 128
            pltpu.make_async_copy(vec_ref.at[pl.ds(q * 128, WW)],
                                  lraw.at[buf, r], load_sems.at[buf]).start()
            return 0

        jax.lax.fori_loop(0, TI, body, 0, unroll=16)

    def _issue_sub(J, s):
        pltpu.make_async_copy(
            sub_ref.at[pl.ds(i0, TI), pl.ds(J * TI, TI)],
            subt.at[s], sub_sems.at[s]).start()

    @pl.when(I == 0)
    def _():
        _issue_loads(0, 0)

    @pl.when(I + 1 < NT)
    def _():
        _issue_loads(I + 1, (I + 1) & 1)

    for dj in range(3):
        @pl.when(I + dj < NT)
        def _():
            _issue_sub(I + dj, (I + dj) & (SD - 1))

    z_ref[...] = jnp.dot(x_ref[...].astype(jnp.bfloat16), w1_ref[...],
                         preferred_element_type=jnp.float32
                         ).astype(jnp.bfloat16)

    def _wait_load(r, _):
        pltpu.make_async_copy(vec_ref.at[pl.ds(0, WW)], lraw.at[I & 1, 0],
                              load_sems.at[I & 1]).wait()
        return 0

    jax.lax.fori_loop(0, TI, _wait_load, 0, unroll=16)

    r2 = jax.lax.broadcasted_iota(jnp.int32, (TI, 1), 0)
    i2 = i0 + r2
    off_w2 = i2 * (N - 1) - (i2 * (i2 - 1)) // 2 - i2 + i0
    sh = off_w2 & 127
    cur = lraw[I & 1].astype(jnp.bfloat16)
    for b in (64, 32, 16, 8, 4, 2, 1):
        rolled = pltpu.roll(cur, WW - b, axis=1)
        cur = jnp.where((sh & b) != 0, rolled, cur)
    s_buf[...] = cur

    rl = jax.lax.broadcasted_iota(jnp.int32, (TI, TI), 0)
    cl = jax.lax.broadcasted_iota(jnp.int32, (TI, TI), 1)
    eye = jnp.where(rl == cl, 1.0, 0.0)
    ones8 = jnp.ones((8, TI), jnp.float32)

    pltpu.make_async_copy(
        sub_ref.at[pl.ds(i0, TI), pl.ds(0, TI)],
        subt.at[I & (SD - 1)], sub_sems.at[I & (SD - 1)]).wait()
    td = s_buf[:, pl.ds(0, TI)].astype(jnp.float32)
    md = jnp.where(cl > rl, td, 0.0)
    pud = jax.nn.sigmoid(md + md.T)
    affd = pud * subt[I & (SD - 1)] + eye

    @pl.when(I > 0)
    def _():
        pltpu.make_async_copy(stpd, p_ref.at[pl.ds(0, TI), pl.ds(0, TI)], diag_sems.at[0]).wait()
        pltpu.make_async_copy(stad, a_ref.at[pl.ds(0, TI), pl.ds(0, TI)], diag_sems.at[1]).wait()

    stpd[...] = pud
    stad[...] = affd.astype(jnp.bfloat16)
    pltpu.make_async_copy(stpd, p_ref.at[pl.ds(i0, TI), pl.ds(i0, TI)],
                          diag_sems.at[0]).start()
    pltpu.make_async_copy(stad, a_ref.at[pl.ds(i0, TI), pl.ds(i0, TI)],
                          diag_sems.at[1]).start()
    acc_tile[...] = affd

    def _tile(J, carry):
        s = J & (SD - 1)
        j0 = J * TI

        @pl.when(J + 2 < NT)
        def _():
            _issue_sub(J + 2, (J + 2) & (SD - 1))

        pltpu.make_async_copy(
            sub_ref.at[pl.ds(i0, TI), pl.ds(0, TI)],
            subt.at[s], sub_sems.at[s]).wait()

        t = s_buf[:, pl.ds((J - I) * TI, TI)].astype(jnp.float32)
        pu = jax.nn.sigmoid(t)
        pu_bf = pu.astype(jnp.bfloat16)
        put = pu_bf.T.astype(jnp.float32)
        af = pu_bf * subt[s].astype(jnp.bfloat16)
        aft = af.T

        @pl.when(J >= I + 1 + SD)
        def _():
            pltpu.make_async_copy(stp.at[s], p_ref.at[pl.ds(0, TI), pl.ds(0, TI)], st_sems.at[0, s]).wait()
            pltpu.make_async_copy(stpt.at[s], p_ref.at[pl.ds(0, TI), pl.ds(0, TI)], st_sems.at[1, s]).wait()
            pltpu.make_async_copy(sta.at[s], a_ref.at[pl.ds(0, TI), pl.ds(0, TI)], st_sems.at[2, s]).wait()
            pltpu.make_async_copy(stat.at[s], a_ref.at[pl.ds(0, TI), pl.ds(0, TI)], st_sems.at[3, s]).wait()

        stp[s] = pu
        stpt[s] = put
        sta[s] = af
        stat[s] = aft
        pltpu.make_async_copy(stp.at[s], p_ref.at[pl.ds(i0, TI), pl.ds(j0, TI)],
                              st_sems.at[0, s]).start()
        pltpu.make_async_copy(stpt.at[s], p_ref.at[pl.ds(j0, TI), pl.ds(i0, TI)],
                              st_sems.at[1, s]).start()
        pltpu.make_async_copy(sta.at[s], a_ref.at[pl.ds(i0, TI), pl.ds(j0, TI)],
                              st_sems.at[2, s]).start()
        pltpu.make_async_copy(stat.at[s], a_ref.at[pl.ds(j0, TI), pl.ds(i0, TI)],
                              st_sems.at[3, s]).start()

        acc_tile[...] += af.astype(jnp.float32)
        cs = jnp.dot(ones8.astype(jnp.bfloat16), af,
                     preferred_element_type=jnp.float32)
        cprev = colacc[pl.ds(J, 1)]
        colacc[pl.ds(J, 1)] = cprev + cs[0:1, :].reshape(1, 1, TI)
        return carry

    jax.lax.fori_loop(I + 1, NT, _tile, 0)

    for jd in range(NT - SD, NT):
        @pl.when(jd >= I + 1)
        def _():
            s = jd & (SD - 1)
            pltpu.make_async_copy(stp.at[s], p_ref.at[pl.ds(0, TI), pl.ds(0, TI)], st_sems.at[0, s]).wait()
            pltpu.make_async_copy(stpt.at[s], p_ref.at[pl.ds(0, TI), pl.ds(0, TI)], st_sems.at[1, s]).wait()
            pltpu.make_async_copy(sta.at[s], a_ref.at[pl.ds(0, TI), pl.ds(0, TI)], st_sems.at[2, s]).wait()
            pltpu.make_async_copy(stat.at[s], a_ref.at[pl.ds(0, TI), pl.ds(0, TI)], st_sems.at[3, s]).wait()

    @pl.when(I == NT - 1)
    def _():
        pltpu.make_async_copy(stpd, p_ref.at[pl.ds(0, TI), pl.ds(0, TI)], diag_sems.at[0]).wait()
        pltpu.make_async_copy(stad, a_ref.at[pl.ds(0, TI), pl.ds(0, TI)], diag_sems.at[1]).wait()

    rs = jnp.dot(acc_tile[...], ones8.T,
                 preferred_element_type=jnp.float32)[:, 0:1]
    deg_ref[...] = (rs.reshape(1, TI) + colacc[pl.ds(I, 1)].reshape(1, TI)
                    ).reshape(1, 1, TI)


def _build_p_and_a(P_vec, sub_adj, x, W1):
    vecpad = jnp.zeros((VEC_PAD_LEN,), jnp.float32).at[1:P_LEN + 1].set(P_vec)
    return pl.pallas_call(
        _build_body,
        grid=(NT,),
        in_specs=[pl.BlockSpec(memory_space=pltpu.MemorySpace.HBM),
                  pl.BlockSpec(memory_space=pltpu.MemorySpace.HBM),
                  pl.BlockSpec((TI, D_IN), lambda i: (i, 0)),
                  pl.BlockSpec((D_IN, D_HID), lambda i: (0, 0))],
        out_specs=[pl.BlockSpec(memory_space=pltpu.MemorySpace.HBM),
                   pl.BlockSpec(memory_space=pltpu.MemorySpace.HBM),
                   pl.BlockSpec((1, 1, TI), lambda i: (i, 0, 0)),
                   pl.BlockSpec((TI, D_HID), lambda i: (i, 0))],
        out_shape=[jax.ShapeDtypeStruct((N, N), jnp.float32),
                   jax.ShapeDtypeStruct((N, N), jnp.bfloat16),
                   jax.ShapeDtypeStruct((NT, 1, TI), jnp.float32),
                   jax.ShapeDtypeStruct((N, D_HID), jnp.bfloat16)],
        scratch_shapes=[
            pltpu.VMEM((2, TI, WW), jnp.float32),
            pltpu.VMEM((TI, WW), jnp.bfloat16),
            pltpu.VMEM((SD, TI, TI), jnp.float32),
            pltpu.VMEM((SD, TI, TI), jnp.float32),
            pltpu.VMEM((SD, TI, TI), jnp.float32),
            pltpu.VMEM((SD, TI, TI), jnp.bfloat16),
            pltpu.VMEM((SD, TI, TI), jnp.bfloat16),
            pltpu.VMEM((TI, TI), jnp.float32),
            pltpu.VMEM((TI, TI), jnp.bfloat16),
            pltpu.VMEM((TI, TI), jnp.float32),
            pltpu.VMEM((NT, 1, TI), jnp.float32),
            pltpu.SemaphoreType.DMA((2,)),
            pltpu.SemaphoreType.DMA((SD,)),
            pltpu.SemaphoreType.DMA((4, SD)),
            pltpu.SemaphoreType.DMA((2,)),
        ],
    )(vecpad, sub_adj, x, W1)


def _layer1_body(a_ref, z_ref, dcol_ref, b1_ref, w2_ref, dinv_ref, g_ref,
                 zs_ref):
    @pl.when(pl.program_id(0) == 0)
    def _():
        zs_ref[...] = (z_ref[...].astype(jnp.float32)
                       * dcol_ref[...]).astype(jnp.bfloat16)

    y = jnp.dot(a_ref[...], zs_ref[...], preferred_element_type=jnp.float32)
    d = dinv_ref[...].reshape(TI, 1)
    h = jnp.maximum(y * d + b1_ref[...], 0.0)
    g = jnp.dot(h, w2_ref[...], preferred_element_type=jnp.float32)
    g_ref[...] = (g * d).astype(jnp.bfloat16)


def _layer2_body(a_ref, g_ref, b2_ref, dinv_ref, out_ref):
    y = jnp.dot(a_ref[...], g_ref[...], preferred_element_type=jnp.float32)
    out_ref[...] = y * dinv_ref[...].reshape(TI, 1) + b2_ref[...]


def kernel(x, P_vec, sub_adj, W1, b1, W2, b2):
    P_used, A, deg_parts, zp = _build_p_and_a(P_vec, sub_adj, x,
                                              W1.astype(jnp.bfloat16))
    deg = deg_parts.reshape(N)
    dinv = jax.lax.rsqrt(deg)
    dcol = dinv.reshape(N, 1)

    g = pl.pallas_call(
        _layer1_body,
        grid=(NT,),
        in_specs=[
            pl.BlockSpec((TI, N), lambda i: (i, 0)),
            pl.BlockSpec((N, D_HID), lambda i: (0, 0)),
            pl.BlockSpec((N, 1), lambda i: (0, 0)),
            pl.BlockSpec((1, D_HID), lambda i: (0, 0)),
            pl.BlockSpec((D_HID, N_CLS), lambda i: (0, 0)),
            pl.BlockSpec((TI,), lambda i: (i,)),
        ],
        out_specs=pl.BlockSpec((TI, N_CLS), lambda i: (i, 0)),
        out_shape=jax.ShapeDtypeStruct((N, N_CLS), jnp.bfloat16),
        scratch_shapes=[pltpu.VMEM((N, D_HID), jnp.bfloat16)],
    )(A, zp, dcol, b1.reshape(1, D_HID), W2, dinv)

    out = pl.pallas_call(
        _layer2_body,
        grid=(NT,),
        in_specs=[
            pl.BlockSpec((TI, N), lambda i: (i, 0)),
            pl.BlockSpec((N, N_CLS), lambda i: (0, 0)),
            pl.BlockSpec((1, N_CLS), lambda i: (0, 0)),
            pl.BlockSpec((TI,), lambda i: (i,)),
        ],
        out_specs=pl.BlockSpec((TI, N_CLS), lambda i: (i, 0)),
        out_shape=jax.ShapeDtypeStruct((N, N_CLS), jnp.float32),
    )(A, g, b2.reshape(1, N_CLS), dinv)

    return (out, P_used)
